# Initial kernel scaffold; baseline (speedup 1.0000x reference)
import functools

import jax
import jax.numpy as jnp
from jax import lax
from jax.experimental import pallas as pl
from jax.experimental.pallas import tpu as pltpu
from jax.experimental.pallas import tpu_sc as plsc

NS = 16
NC = 2
NW = NC * NS
DEG_COLS = 8


def _mesh():
    return plsc.VectorSubcoreMesh(core_axis_name="c", subcore_axis_name="s")


def _deg_kernel(NP, NCH, CH):
    rpt = NP // NS

    @functools.partial(
        pl.kernel,
        out_type=jax.ShapeDtypeStruct((NC, NP, DEG_COLS), jnp.float32),
        mesh=_mesh(),
        scratch_types=[
            pltpu.VMEM((NCH, CH), jnp.int32),
            pltpu.VMEM((CH, DEG_COLS), jnp.float32),
            pltpu.VMEM_SHARED((NP, DEG_COLS), jnp.float32),
        ],
    )
    def k(dstT, ones8, zeros8, out, dstv, ones_v, acc):
        c = lax.axis_index("c")
        s = lax.axis_index("s")
        wid = c * NS + s
        r0 = s * rpt
        pltpu.sync_copy(zeros8.at[pl.ds(r0, rpt)], acc.at[pl.ds(r0, rpt)])
        pltpu.sync_copy(dstT.at[wid], dstv)
        pltpu.sync_copy(ones8, ones_v)
        plsc.subcore_barrier()

        def body(j, carry):
            pltpu.sync_copy(ones_v, acc.at[dstv.at[j]], add=True)
            return carry

        lax.fori_loop(0, NCH, body, 0)
        plsc.subcore_barrier()
        pltpu.sync_copy(acc.at[pl.ds(r0, rpt)], out.at[c, pl.ds(r0, rpt)])

    return k


def _agg_kernel(Nn, D, NCH, CH):
    rpt = Nn // NS

    @functools.partial(
        pl.kernel,
        out_type=jax.ShapeDtypeStruct((NC, Nn, D), jnp.float32),
        mesh=_mesh(),
        scratch_types=[
            pltpu.VMEM((NCH, CH), jnp.int32),
            pltpu.VMEM((NCH, CH), jnp.int32),
            pltpu.VMEM((CH, D), jnp.float32),
            pltpu.VMEM_SHARED((Nn, D), jnp.float32),
            pltpu.SemaphoreType.DMA,
        ],
    )
    def k(hs, srcT, dstT, out, srcv, dstv, buf, acc, sem):
        c = lax.axis_index("c")
        s = lax.axis_index("s")
        wid = c * NS + s
        r0 = s * rpt
        pltpu.sync_copy(hs.at[pl.ds(r0, rpt)], acc.at[pl.ds(r0, rpt)])
        pltpu.sync_copy(srcT.at[wid], srcv)
        pltpu.sync_copy(dstT.at[wid], dstv)
        plsc.subcore_barrier()

        def body(j, carry):
            pltpu.async_copy(hs.at[srcv.at[j]], buf, sem).wait()
            pltpu.sync_copy(buf, acc.at[dstv.at[j]], add=True)
            return carry

        lax.fori_loop(0, NCH, body, 0)
        plsc.subcore_barrier()
        pltpu.sync_copy(acc.at[pl.ds(r0, rpt)], out.at[c, pl.ds(r0, rpt)])

    return k


def _stage0_call(x, W1, d0, d1, RB):
    Nn, Din = x.shape
    Dh = W1.shape[1]
    grid = (Nn // RB,)

    def body(x_r, w_r, d0_r, d1_r, hs_r, dinv_r):
        deg = d0_r[...] + d1_r[...] + 1.0
        dinv = lax.rsqrt(deg)
        dinv_r[...] = dinv
        h = jnp.dot(x_r[...], w_r[...], preferred_element_type=jnp.float32)
        hs_r[...] = dinv * h

    return pl.pallas_call(
        body,
        grid=grid,
        in_specs=[
            pl.BlockSpec((RB, Din), lambda i: (i, 0)),
            pl.BlockSpec((Din, Dh), lambda i: (0, 0)),
            pl.BlockSpec((RB, 1), lambda i: (i, 0)),
            pl.BlockSpec((RB, 1), lambda i: (i, 0)),
        ],
        out_specs=[
            pl.BlockSpec((RB, Dh), lambda i: (i, 0)),
            pl.BlockSpec((RB, 1), lambda i: (i, 0)),
        ],
        out_shape=[
            jax.ShapeDtypeStruct((Nn, Dh), jnp.float32),
            jax.ShapeDtypeStruct((Nn, 1), jnp.float32),
        ],
    )(x, W1, d0, d1)


def _stagemid_call(p0, p1, hs, dinv, b, W, RB):
    Nn, D = hs.shape
    Dh = W.shape[1]
    grid = (Nn // RB,)

    def body(p0_r, p1_r, hs_r, dinv_r, b_r, w_r, o_r):
        dv = dinv_r[...]
        t = jnp.maximum(dv * (p0_r[...] + p1_r[...] - hs_r[...]) + b_r[...], 0.0)
        o_r[...] = dv * jnp.dot(t, w_r[...], preferred_element_type=jnp.float32)

    return pl.pallas_call(
        body,
        grid=grid,
        in_specs=[
            pl.BlockSpec((RB, D), lambda i: (i, 0)),
            pl.BlockSpec((RB, D), lambda i: (i, 0)),
            pl.BlockSpec((RB, D), lambda i: (i, 0)),
            pl.BlockSpec((RB, 1), lambda i: (i, 0)),
            pl.BlockSpec((1, D), lambda i: (0, 0)),
            pl.BlockSpec((D, Dh), lambda i: (0, 0)),
        ],
        out_specs=pl.BlockSpec((RB, Dh), lambda i: (i, 0)),
        out_shape=jax.ShapeDtypeStruct((Nn, Dh), jnp.float32),
    )(p0, p1, hs, dinv, b, W)


def _stagefinal_call(p0, p1, hs, dinv, b, W, bo, RB):
    Nn, D = hs.shape
    Do = W.shape[1]
    grid = (Nn // RB,)

    def body(p0_r, p1_r, hs_r, dinv_r, b_r, w_r, bo_r, o_r):
        dv = dinv_r[...]
        t = jnp.maximum(dv * (p0_r[...] + p1_r[...] - hs_r[...]) + b_r[...], 0.0)
        o_r[...] = jnp.dot(t, w_r[...], preferred_element_type=jnp.float32) + bo_r[...]

    return pl.pallas_call(
        body,
        grid=grid,
        in_specs=[
            pl.BlockSpec((RB, D), lambda i: (i, 0)),
            pl.BlockSpec((RB, D), lambda i: (i, 0)),
            pl.BlockSpec((RB, D), lambda i: (i, 0)),
            pl.BlockSpec((RB, 1), lambda i: (i, 0)),
            pl.BlockSpec((1, D), lambda i: (0, 0)),
            pl.BlockSpec((D, Do), lambda i: (0, 0)),
            pl.BlockSpec((1, Do), lambda i: (0, 0)),
        ],
        out_specs=pl.BlockSpec((RB, Do), lambda i: (i, 0)),
        out_shape=jax.ShapeDtypeStruct((Nn, Do), jnp.float32),
    )(p0, p1, hs, dinv, b, W, bo)


def kernel(x, edge_index, W1, b1, W2, b2, Wout, bout):
    Nn, Din = x.shape
    E = edge_index.shape[1]
    per_tile = E // NW
    assert per_tile * NW == E
    CH = 80
    NCH = per_tile // CH
    assert NCH * CH == per_tile

    src = edge_index[0].reshape(NW, NCH, CH)
    dst = edge_index[1].reshape(NW, NCH, CH)

    NP = ((Nn + 8 * NS - 1) // (8 * NS)) * (8 * NS)
    ones8 = jnp.zeros((CH, DEG_COLS), jnp.float32).at[:, 0].set(1.0)
    zeros8 = jnp.zeros((NP, DEG_COLS), jnp.float32)
    degp = _deg_kernel(NP, NCH, CH)(dst, ones8, zeros8)
    d0 = degp[0, :Nn, 0:1]
    d1 = degp[1, :Nn, 0:1]

    RB = 2000
    hs1, dinv = _stage0_call(x, W1, d0, d1, RB)

    agg = _agg_kernel(Nn, W1.shape[1], NCH, CH)
    p = agg(hs1, src, dst)
    hs2 = _stagemid_call(p[0], p[1], hs1, dinv, b1.reshape(1, -1), W2, RB)
    q = agg(hs2, src, dst)
    out = _stagefinal_call(
        q[0], q[1], hs2, dinv, b2.reshape(1, -1), Wout, bout.reshape(1, -1), RB
    )
    return out


# speedup vs baseline: 12.2669x; 12.2669x over previous
import functools

import jax
import jax.numpy as jnp
from jax import lax
from jax.experimental import pallas as pl
from jax.experimental.pallas import tpu as pltpu
from jax.experimental.pallas import tpu_sc as plsc

NS = 16
NC = 2
NW = NC * NS
CH = 80
D = 128


def _mesh():
    return plsc.VectorSubcoreMesh(core_axis_name="c", subcore_axis_name="s")


def _deg_kernel(NP, NCH):
    rpt = NP // NS

    @functools.partial(
        pl.kernel,
        out_type=jax.ShapeDtypeStruct((NC, NP, D), jnp.float32),
        mesh=_mesh(),
        scratch_types=[
            pltpu.VMEM((CH,), jnp.int32),
            pltpu.VMEM((CH, D), jnp.float32),
            pltpu.VMEM_SHARED((NP, D), jnp.float32),
            pltpu.SemaphoreType.DMA,
        ],
    )
    def k(dstT, ones, zeros, out, dstv, ones_v, acc, sem):
        c = lax.axis_index("c")
        s = lax.axis_index("s")
        wid = c * NS + s
        r0 = s * rpt
        pltpu.sync_copy(zeros.at[pl.ds(r0, rpt)], acc.at[pl.ds(r0, rpt)])
        pltpu.sync_copy(ones, ones_v)
        plsc.subcore_barrier()

        def body(j, carry):
            base = wid * (NCH * CH) + j * CH
            pltpu.sync_copy(dstT.at[pl.ds(base, CH)], dstv)
            pltpu.async_copy(ones_v, acc.at[dstv], sem, add=True).wait()
            return carry

        lax.fori_loop(0, NCH, body, 0)
        plsc.subcore_barrier()
        pltpu.sync_copy(acc.at[pl.ds(r0, rpt)], out.at[c, pl.ds(r0, rpt)])

    return k


def _agg_kernel(NP, NCH):
    rpt = NP // NS

    @functools.partial(
        pl.kernel,
        out_type=jax.ShapeDtypeStruct((NC, NP, D), jnp.float32),
        mesh=_mesh(),
        scratch_types=[
            pltpu.VMEM((CH,), jnp.int32),
            pltpu.VMEM((CH,), jnp.int32),
            pltpu.VMEM((CH, D), jnp.float32),
            pltpu.VMEM_SHARED((NP, D), jnp.float32),
            pltpu.SemaphoreType.DMA,
            pltpu.SemaphoreType.DMA,
        ],
    )
    def k(hs, srcF, dstF, out, srcv, dstv, buf, acc, sem, sem2):
        c = lax.axis_index("c")
        s = lax.axis_index("s")
        wid = c * NS + s
        r0 = s * rpt
        pltpu.sync_copy(hs.at[pl.ds(r0, rpt)], acc.at[pl.ds(r0, rpt)])
        plsc.subcore_barrier()

        def body(j, carry):
            base = wid * (NCH * CH) + j * CH
            pltpu.sync_copy(srcF.at[pl.ds(base, CH)], srcv)
            pltpu.sync_copy(dstF.at[pl.ds(base, CH)], dstv)
            pltpu.async_copy(hs.at[srcv], buf, sem).wait()
            pltpu.async_copy(buf, acc.at[dstv], sem2, add=True).wait()
            return carry

        lax.fori_loop(0, NCH, body, 0)
        plsc.subcore_barrier()
        pltpu.sync_copy(acc.at[pl.ds(r0, rpt)], out.at[c, pl.ds(r0, rpt)])

    return k


def _stage0_call(x, W1, d0, d1, NP, RB):
    Nn, Din = x.shape
    Dh = W1.shape[1]
    grid = (NP // RB,)

    def body(x_r, w_r, d0_r, d1_r, hs_r, dinv_r):
        deg = d0_r[...] + d1_r[...] + 1.0
        dinv = lax.rsqrt(deg)
        dinv_r[...] = dinv
        h = jnp.dot(x_r[...], w_r[...], preferred_element_type=jnp.float32)
        hs_r[...] = dinv * h

    return pl.pallas_call(
        body,
        grid=grid,
        in_specs=[
            pl.BlockSpec((RB, Din), lambda i: (i, 0)),
            pl.BlockSpec((Din, Dh), lambda i: (0, 0)),
            pl.BlockSpec((RB, 1), lambda i: (i, 0)),
            pl.BlockSpec((RB, 1), lambda i: (i, 0)),
        ],
        out_specs=[
            pl.BlockSpec((RB, Dh), lambda i: (i, 0)),
            pl.BlockSpec((RB, 1), lambda i: (i, 0)),
        ],
        out_shape=[
            jax.ShapeDtypeStruct((NP, Dh), jnp.float32),
            jax.ShapeDtypeStruct((NP, 1), jnp.float32),
        ],
    )(x, W1, d0, d1)


def _stagemid_call(p0, p1, hs, dinv, b, W, RB):
    Nn, Dh0 = hs.shape
    Dh = W.shape[1]
    grid = (Nn // RB,)

    def body(p0_r, p1_r, hs_r, dinv_r, b_r, w_r, o_r):
        dv = dinv_r[...]
        t = jnp.maximum(dv * (p0_r[...] + p1_r[...] - hs_r[...]) + b_r[...], 0.0)
        o_r[...] = dv * jnp.dot(t, w_r[...], preferred_element_type=jnp.float32)

    return pl.pallas_call(
        body,
        grid=grid,
        in_specs=[
            pl.BlockSpec((RB, Dh0), lambda i: (i, 0)),
            pl.BlockSpec((RB, Dh0), lambda i: (i, 0)),
            pl.BlockSpec((RB, Dh0), lambda i: (i, 0)),
            pl.BlockSpec((RB, 1), lambda i: (i, 0)),
            pl.BlockSpec((1, Dh0), lambda i: (0, 0)),
            pl.BlockSpec((Dh0, Dh), lambda i: (0, 0)),
        ],
        out_specs=pl.BlockSpec((RB, Dh), lambda i: (i, 0)),
        out_shape=jax.ShapeDtypeStruct((Nn, Dh), jnp.float32),
    )(p0, p1, hs, dinv, b, W)


def _stagefinal_call(p0, p1, hs, dinv, b, W, bo, Nn, RB):
    Dh0 = hs.shape[1]
    Do = W.shape[1]
    grid = (Nn // RB,)

    def body(p0_r, p1_r, hs_r, dinv_r, b_r, w_r, bo_r, o_r):
        dv = dinv_r[...]
        t = jnp.maximum(dv * (p0_r[...] + p1_r[...] - hs_r[...]) + b_r[...], 0.0)
        o_r[...] = jnp.dot(t, w_r[...], preferred_element_type=jnp.float32) + bo_r[...]

    return pl.pallas_call(
        body,
        grid=grid,
        in_specs=[
            pl.BlockSpec((RB, Dh0), lambda i: (i, 0)),
            pl.BlockSpec((RB, Dh0), lambda i: (i, 0)),
            pl.BlockSpec((RB, Dh0), lambda i: (i, 0)),
            pl.BlockSpec((RB, 1), lambda i: (i, 0)),
            pl.BlockSpec((1, Dh0), lambda i: (0, 0)),
            pl.BlockSpec((Dh0, Do), lambda i: (0, 0)),
            pl.BlockSpec((1, Do), lambda i: (0, 0)),
        ],
        out_specs=pl.BlockSpec((RB, Do), lambda i: (i, 0)),
        out_shape=jax.ShapeDtypeStruct((Nn, Do), jnp.float32),
    )(p0, p1, hs, dinv, b, W, bo)


def kernel(x, edge_index, W1, b1, W2, b2, Wout, bout):
    Nn, Din = x.shape
    E = edge_index.shape[1]
    per_tile = E // NW
    assert per_tile * NW == E
    NCH = per_tile // CH
    assert NCH * CH == per_tile

    RBP = 2048
    NP = ((Nn + RBP - 1) // RBP) * RBP

    src_f = edge_index[0]
    dst_f = edge_index[1]

    ones = jnp.zeros((CH, D), jnp.float32).at[:, 0].set(1.0)
    zeros = jnp.zeros((NP, D), jnp.float32)
    degp = _deg_kernel(NP, NCH)(dst_f, ones, zeros)
    d0 = degp[0, :, 0:1]
    d1 = degp[1, :, 0:1]

    hs1, dinv = _stage0_call(x, W1, d0, d1, NP, RBP)

    agg = _agg_kernel(NP, NCH)
    p = agg(hs1, src_f, dst_f)
    hs2 = _stagemid_call(p[0], p[1], hs1, dinv, b1.reshape(1, -1), W2, RBP)
    q = agg(hs2, src_f, dst_f)
    RB = 2000
    assert Nn % RB == 0
    out = _stagefinal_call(
        q[0], q[1], hs2, dinv, b2.reshape(1, -1), Wout, bout.reshape(1, -1), Nn, RB
    )
    return out
